# Initial kernel scaffold; baseline (speedup 1.0000x reference)
import jax
import jax.numpy as jnp
from jax.experimental import pallas as pl


def kernel(x, grid, W1, W2, R1, R2, R3):
    raise NotImplementedError("write your pallas kernel here")


# speedup vs baseline: 70.4813x; 70.4813x over previous
import functools

import jax
import jax.numpy as jnp
import numpy as np
from jax.experimental import pallas as pl

_L = 8
_F = 2
_T = 1024
_B_G = float(np.exp(np.log(20 * 0.5 / 2) / (_L - 1)))
_RES = [int(np.floor(2 * _B_G**l)) for l in range(_L)]
_S = [r + 1 for r in _RES]
_SX = int(sum(_S))
_C2 = [s * s * _F for s in _S]
_SC2 = int(sum(_C2))
_XOFF = np.concatenate([[0], np.cumsum(_S)]).astype(np.int64)
_COFF = np.concatenate([[0], np.cumsum(_C2)]).astype(np.int64)


def _build_tables():
    gidx = np.zeros((_SX, _SC2), np.int64)
    gmask = np.zeros((_SX, _SC2), np.float32)
    pye = np.zeros((_SX, _SC2), np.float32)
    pze = np.zeros((_SX, _SC2), np.float32)
    csel = np.zeros((_SC2, _L * _F), np.float32)
    res_row = np.zeros((1, _SX), np.float32)
    iota_row = np.zeros((1, _SX), np.float32)
    for l, s in enumerate(_S):
        ar = np.arange(s)
        I, J, K = np.meshgrid(ar, ar, ar, indexing="ij")
        h = (
            (I.astype(np.uint32) * np.uint32(1))
            ^ (J.astype(np.uint32) * np.uint32(2654435761))
            ^ (K.astype(np.uint32) * np.uint32(805459861))
        ) % np.uint32(_T)
        h = h.astype(np.int64)
        res_row[0, _XOFF[l] : _XOFF[l + 1]] = float(_RES[l])
        iota_row[0, _XOFF[l] : _XOFF[l + 1]] = ar.astype(np.float32)
        for f in range(_F):
            rows = _XOFF[l] + I
            cols = _COFF[l] + (J * s + K) * _F + f
            gidx[rows, cols] = (l * _T + h) * _F + f
            gmask[rows, cols] = 1.0
            pye[_XOFF[l] + J, cols] = 1.0
            pze[_XOFF[l] + K, cols] = 1.0
            csel[cols, 2 * l + f] = 1.0
    return gidx, gmask, pye, pze, csel, res_row, iota_row


_GIDX, _GMASK, _PYE, _PZE, _CSEL, _RES_ROW, _IOTA_ROW = _build_tables()


def _body(x_ref, dxa_ref, pye_ref, pze_ref, cw1_ref, w2_ref, r1_ref,
          r2_ref, r3_ref, res_ref, iota_ref, o_ref):
    x = x_ref[...]
    res = res_ref[...]
    iota = iota_ref[...]
    f32 = jnp.float32
    hx = jnp.maximum(0.0, 1.0 - jnp.abs(x[:, 0:1] * res - iota))
    hy = jnp.maximum(0.0, 1.0 - jnp.abs(x[:, 1:2] * res - iota))
    hz = jnp.maximum(0.0, 1.0 - jnp.abs(x[:, 2:3] * res - iota))
    e = jnp.dot(hx, dxa_ref[...], preferred_element_type=f32)
    hy_e = jnp.dot(hy, pye_ref[...], preferred_element_type=f32)
    hz_e = jnp.dot(hz, pze_ref[...], preferred_element_type=f32)
    em = e * hy_e * hz_e
    t = jnp.maximum(jnp.dot(em, cw1_ref[...], preferred_element_type=f32), 0.0)
    h2 = jnp.dot(t, w2_ref[...], preferred_element_type=f32)
    r = jnp.maximum(jnp.dot(h2, r1_ref[...], preferred_element_type=f32), 0.0)
    r = jnp.maximum(jnp.dot(r, r2_ref[...], preferred_element_type=f32), 0.0)
    o_ref[...] = jax.nn.sigmoid(
        jnp.dot(r, r3_ref[...], preferred_element_type=f32))


@functools.partial(jax.jit, static_argnames=("nb", "interpret"))
def _run(x, dxa, pye, pze, cw1, w2, r1, r2, r3, nb=1024, interpret=False):
    n = x.shape[0]
    res_row = jnp.asarray(_RES_ROW)
    iota_row = jnp.asarray(_IOTA_ROW)
    full = lambda a: pl.BlockSpec(a.shape, lambda i: (0,) * a.ndim)
    out = pl.pallas_call(
        _body,
        grid=(n // nb,),
        in_specs=[
            pl.BlockSpec((nb, 3), lambda i: (i, 0)),
            full(dxa), full(pye), full(pze), full(cw1), full(w2),
            full(r1), full(r2), full(r3), full(res_row), full(iota_row),
        ],
        out_specs=pl.BlockSpec((nb, 1), lambda i: (i, 0)),
        out_shape=jax.ShapeDtypeStruct((n, 1), jnp.float32),
        interpret=interpret,
    )(x, dxa, pye, pze, cw1, w2, r1, r2, r3, res_row, iota_row)
    return out


def kernel(x, grid, W1, W2, R1, R2, R3, interpret=False):
    grid_flat = grid.reshape(-1)
    dxa = grid_flat[jnp.asarray(_GIDX)] * jnp.asarray(_GMASK)
    cw1 = jnp.asarray(_CSEL) @ W1
    return _run(x, dxa, jnp.asarray(_PYE), jnp.asarray(_PZE), cw1,
                W2, R1, R2, R3, interpret=interpret)


# speedup vs baseline: 72.3920x; 1.0271x over previous
import functools

import jax
import jax.numpy as jnp
import numpy as np
from jax.experimental import pallas as pl

_L = 8
_F = 2
_T = 1024
_B_G = float(np.exp(np.log(20 * 0.5 / 2) / (_L - 1)))
_RES = [int(np.floor(2 * _B_G**l)) for l in range(_L)]
_S = [r + 1 for r in _RES]
_SX = int(sum(_S))
_C2 = [s * s * _F for s in _S]
_SC2 = int(sum(_C2))
_XOFF = np.concatenate([[0], np.cumsum(_S)]).astype(np.int64)
_COFF = np.concatenate([[0], np.cumsum(_C2)]).astype(np.int64)


def _build_tables():
    gidx = np.zeros((_SX, _SC2), np.int64)
    gmask = np.zeros((_SX, _SC2), np.float32)
    pye = np.zeros((_SX, _SC2), np.float32)
    pze = np.zeros((_SX, _SC2), np.float32)
    csel = np.zeros((_SC2, _L * _F), np.float32)
    res_row = np.zeros((1, _SX), np.float32)
    iota_row = np.zeros((1, _SX), np.float32)
    for l, s in enumerate(_S):
        ar = np.arange(s)
        I, J, K = np.meshgrid(ar, ar, ar, indexing="ij")
        h = (
            (I.astype(np.uint32) * np.uint32(1))
            ^ (J.astype(np.uint32) * np.uint32(2654435761))
            ^ (K.astype(np.uint32) * np.uint32(805459861))
        ) % np.uint32(_T)
        h = h.astype(np.int64)
        res_row[0, _XOFF[l] : _XOFF[l + 1]] = float(_RES[l])
        iota_row[0, _XOFF[l] : _XOFF[l + 1]] = ar.astype(np.float32)
        for f in range(_F):
            rows = _XOFF[l] + I
            cols = _COFF[l] + (J * s + K) * _F + f
            gidx[rows, cols] = (l * _T + h) * _F + f
            gmask[rows, cols] = 1.0
            pye[_XOFF[l] + J, cols] = 1.0
            pze[_XOFF[l] + K, cols] = 1.0
            csel[cols, 2 * l + f] = 1.0
    return gidx, gmask, pye, pze, csel, res_row, iota_row


_GIDX, _GMASK, _PYE, _PZE, _CSEL, _RES_ROW, _IOTA_ROW = _build_tables()


def _body(x_ref, dxa_ref, pye_ref, pze_ref, cw1_ref, w21_ref,
          r2_ref, r3_ref, res_ref, iota_ref, o_ref):
    x = x_ref[...]
    res = res_ref[...]
    iota = iota_ref[...]
    f32 = jnp.float32
    hx = jnp.maximum(0.0, 1.0 - jnp.abs(x[:, 0:1] * res - iota))
    hy = jnp.maximum(0.0, 1.0 - jnp.abs(x[:, 1:2] * res - iota))
    hz = jnp.maximum(0.0, 1.0 - jnp.abs(x[:, 2:3] * res - iota))
    e = jnp.dot(hx, dxa_ref[...], preferred_element_type=f32)
    hy_e = jnp.dot(hy, pye_ref[...], preferred_element_type=f32)
    hz_e = jnp.dot(hz, pze_ref[...], preferred_element_type=f32)
    em = e * hy_e * hz_e
    t = jnp.maximum(jnp.dot(em, cw1_ref[...], preferred_element_type=f32), 0.0)
    r = jnp.maximum(jnp.dot(t, w21_ref[...], preferred_element_type=f32), 0.0)
    r = jnp.maximum(jnp.dot(r, r2_ref[...], preferred_element_type=f32), 0.0)
    o_ref[...] = jax.nn.sigmoid(
        jnp.dot(r, r3_ref[...], preferred_element_type=f32))


@functools.partial(jax.jit, static_argnames=("nb", "interpret"))
def _run(x, dxa, pye, pze, cw1, w21, r2, r3, nb=1024, interpret=False):
    n = x.shape[0]
    res_row = jnp.asarray(_RES_ROW)
    iota_row = jnp.asarray(_IOTA_ROW)
    full = lambda a: pl.BlockSpec(a.shape, lambda i: (0,) * a.ndim)
    out = pl.pallas_call(
        _body,
        grid=(n // nb,),
        in_specs=[
            pl.BlockSpec((nb, 3), lambda i: (i, 0)),
            full(dxa), full(pye), full(pze), full(cw1), full(w21),
            full(r2), full(r3), full(res_row), full(iota_row),
        ],
        out_specs=pl.BlockSpec((nb, 1), lambda i: (i, 0)),
        out_shape=jax.ShapeDtypeStruct((n, 1), jnp.float32),
        interpret=interpret,
    )(x, dxa, pye, pze, cw1, w21, r2, r3, res_row, iota_row)
    return out


def kernel(x, grid, W1, W2, R1, R2, R3, interpret=False):
    grid_flat = grid.reshape(-1)
    dxa = grid_flat[jnp.asarray(_GIDX)] * jnp.asarray(_GMASK)
    cw1 = jnp.asarray(_CSEL) @ W1
    w21 = W2 @ R1
    return _run(x, dxa, jnp.asarray(_PYE), jnp.asarray(_PZE), cw1,
                w21, R2, R3, interpret=interpret)


# speedup vs baseline: 176.8687x; 2.4432x over previous
import functools

import jax
import jax.numpy as jnp
import numpy as np
from jax import lax
from jax.experimental import pallas as pl
from jax.experimental.pallas import tpu as pltpu
from jax.experimental.pallas import tpu_sc as plsc

_L = 8
_T = 1024
_B_G = float(np.exp(np.log(20 * 0.5 / 2) / (_L - 1)))
_RES = [int(np.floor(2 * _B_G**l)) for l in range(_L)]
_C1 = np.int32(np.uint32(2654435761).astype(np.int32))
_C2 = np.int32(805459861)

_NW = 32
_CNK = 2048


def _sc_encode(xt, grid_flat):
    n = xt.shape[1]
    npts = n // _NW
    nchunk = npts // _CNK
    mesh = plsc.VectorSubcoreMesh(core_axis_name="c", subcore_axis_name="s")

    @functools.partial(
        pl.kernel,
        out_type=jax.ShapeDtypeStruct((16, n), jnp.float32),
        mesh=mesh,
        scratch_types=[
            pltpu.VMEM((_L * _T * 2,), jnp.float32),
            pltpu.VMEM((3, _CNK), jnp.float32),
            pltpu.VMEM((16, _CNK), jnp.float32),
        ],
        compiler_params=pltpu.CompilerParams(needs_layout_passes=False),
    )
    def enc(xt_hbm, grid_hbm, ft_hbm, gv, xv, fv):
        wid = lax.axis_index("s") * 2 + lax.axis_index("c")
        pltpu.sync_copy(grid_hbm, gv)
        base0 = wid * npts

        def chunk_body(ci, _):
            base = base0 + ci * _CNK
            pltpu.sync_copy(xt_hbm.at[:, pl.ds(base, _CNK)], xv)

            def pt_body(i, _):
                sl = pl.ds(i * 16, 16)
                xs = xv[0, sl]
                ys = xv[1, sl]
                zs = xv[2, sl]
                for l in range(_L):
                    res = float(_RES[l])
                    px = xs * res
                    py = ys * res
                    pz = zs * res
                    ix = px.astype(jnp.int32)
                    iy = py.astype(jnp.int32)
                    iz = pz.astype(jnp.int32)
                    wx1 = px - ix.astype(jnp.float32)
                    wy1 = py - iy.astype(jnp.float32)
                    wz1 = pz - iz.astype(jnp.float32)
                    hy = (iy * _C1, (iy + 1) * _C1)
                    hz = (iz * _C2, (iz + 1) * _C2)
                    hx = (ix, ix + 1)
                    wxs = (1.0 - wx1, wx1)
                    wys = (1.0 - wy1, wy1)
                    wzs = (1.0 - wz1, wz1)
                    acc0 = None
                    acc1 = None
                    for dx in (0, 1):
                        for dy in (0, 1):
                            hxy = hx[dx] ^ hy[dy]
                            wxy = wxs[dx] * wys[dy]
                            for dz in (0, 1):
                                h = (hxy ^ hz[dz]) & (_T - 1)
                                idx = h * 2 + (l * _T * 2)
                                g0 = plsc.load_gather(gv, [idx])
                                g1 = plsc.load_gather(gv, [idx + 1])
                                w = wxy * wzs[dz]
                                if acc0 is None:
                                    acc0 = w * g0
                                    acc1 = w * g1
                                else:
                                    acc0 = acc0 + w * g0
                                    acc1 = acc1 + w * g1
                    fv[2 * l, sl] = acc0
                    fv[2 * l + 1, sl] = acc1
                return 0

            lax.fori_loop(0, _CNK // 16, pt_body, 0)
            pltpu.sync_copy(fv, ft_hbm.at[:, pl.ds(base, _CNK)])
            return 0

        lax.fori_loop(0, nchunk, chunk_body, 0)

    return enc(xt, grid_flat)


def _mlp_body(ft_ref, w1_ref, w21_ref, r2_ref, r3_ref, o_ref):
    f32 = jnp.float32
    ft = ft_ref[...]
    t = lax.dot_general(ft, w1_ref[...], (((0,), (0,)), ((), ())),
                        preferred_element_type=f32)
    t = jnp.maximum(t, 0.0)
    r = jnp.maximum(jnp.dot(t, w21_ref[...], preferred_element_type=f32), 0.0)
    r = jnp.maximum(jnp.dot(r, r2_ref[...], preferred_element_type=f32), 0.0)
    o_ref[...] = jax.nn.sigmoid(
        jnp.dot(r, r3_ref[...], preferred_element_type=f32))


@functools.partial(jax.jit, static_argnames=("nb",))
def _run(x, grid, W1, w21, R2, R3, nb=2048):
    n = x.shape[0]
    xt = x.T
    ft = _sc_encode(xt, grid.reshape(-1))
    full = lambda a: pl.BlockSpec(a.shape, lambda i: (0,) * a.ndim)
    out = pl.pallas_call(
        _mlp_body,
        grid=(n // nb,),
        in_specs=[
            pl.BlockSpec((16, nb), lambda i: (0, i)),
            full(W1), full(w21), full(R2), full(R3),
        ],
        out_specs=pl.BlockSpec((nb, 1), lambda i: (i, 0)),
        out_shape=jax.ShapeDtypeStruct((n, 1), jnp.float32),
    )(ft, W1, w21, R2, R3)
    return out


def kernel(x, grid, W1, W2, R1, R2, R3):
    w21 = W2 @ R1
    return _run(x, grid, W1, w21, R2, R3)
